# Initial kernel scaffold; baseline (speedup 1.0000x reference)
import jax
import jax.numpy as jnp
from jax.experimental import pallas as pl


def kernel(atomic_num, embedding_table):
    raise NotImplementedError("write your pallas kernel here")


# speedup vs baseline: 1.4240x; 1.4240x over previous
import functools

import jax
import jax.numpy as jnp
from jax import lax
from jax.experimental import pallas as pl
from jax.experimental.pallas import tpu as pltpu
from jax.experimental.pallas import tpu_sc as plsc

N = 100000
D = 128
NW = 32
BLK = 128
NFULL = N // BLK
REM = N - NFULL * BLK
TRIPS = (NFULL + NW - 1) // NW


def _make_kernel():
    mesh = plsc.VectorSubcoreMesh(core_axis_name="c", subcore_axis_name="s")

    @functools.partial(
        pl.kernel,
        mesh=mesh,
        out_type=jax.ShapeDtypeStruct((N, D), jnp.float32),
        scratch_types=[
            pltpu.VMEM((BLK,), jnp.int32),
            pltpu.VMEM((BLK, D), jnp.float32),
            pltpu.VMEM((REM,), jnp.int32),
            pltpu.VMEM((REM, D), jnp.float32),
            pltpu.SemaphoreType.DMA,
        ],
    )
    def k(table_hbm, idx_hbm, out_hbm, idx_v, rows_v, idx_r, rows_r, sem):
        wid = lax.axis_index("s") * 2 + lax.axis_index("c")

        def body(t, carry):
            b = t * NW + wid

            @pl.when(b < NFULL)
            def _():
                off = b * BLK
                pltpu.sync_copy(idx_hbm.at[pl.ds(off, BLK)], idx_v)
                pltpu.async_copy(table_hbm.at[idx_v], rows_v, sem).wait()
                pltpu.sync_copy(rows_v, out_hbm.at[pl.ds(off, BLK)])

            return carry

        lax.fori_loop(0, TRIPS, body, 0)

        @pl.when(wid == NW - 1)
        def _():
            off = NFULL * BLK
            pltpu.sync_copy(idx_hbm.at[pl.ds(off, REM)], idx_r)
            pltpu.async_copy(table_hbm.at[idx_r], rows_r, sem).wait()
            pltpu.sync_copy(rows_r, out_hbm.at[pl.ds(off, REM)])

    return k


_kernel = _make_kernel()


def kernel(atomic_num, embedding_table):
    idx = atomic_num.astype(jnp.int32)
    return _kernel(embedding_table, idx)


# speedup vs baseline: 5.6771x; 3.9868x over previous
import functools

import jax
import jax.numpy as jnp
from jax import lax
from jax.experimental import pallas as pl
from jax.experimental.pallas import tpu as pltpu
from jax.experimental.pallas import tpu_sc as plsc

N = 100000
D = 128
V = 95
NW = 32
B_MAIN = 3128
B_LAST = N - 31 * B_MAIN
BLK = 128
NBUF = 6
LOOK = 5


def _blocks(total):
    full = total // BLK
    rem = total - full * BLK
    out = [(j * BLK, BLK) for j in range(full)]
    if rem:
        out.append((full * BLK, rem))
    return out


def _make_kernel():
    mesh = plsc.VectorSubcoreMesh(core_axis_name="c", subcore_axis_name="s")

    @functools.partial(
        pl.kernel,
        mesh=mesh,
        out_type=jax.ShapeDtypeStruct((N, D), jnp.float32),
        scratch_types=[
            pltpu.VMEM_SHARED((V, D), jnp.float32),
            pltpu.VMEM((B_MAIN,), jnp.int32),
            pltpu.VMEM((NBUF, BLK, D), jnp.float32),
            pltpu.SemaphoreType.DMA((NBUF,)),
            pltpu.SemaphoreType.DMA((NBUF,)),
        ],
    )
    def k(table_hbm, idx_hbm, out_hbm, table_sh, idx_v, rows, sem_g, sem_w):
        cid = lax.axis_index("c")
        sid = lax.axis_index("s")
        wid = sid * 2 + cid

        @pl.when(sid == 0)
        def _():
            pltpu.sync_copy(table_hbm, table_sh)

        plsc.subcore_barrier()

        def ring(base, nrows):
            blocks = _blocks(nrows)
            K = len(blocks)

            def g_copy(kk):
                off, sz = blocks[kk]
                b = kk % NBUF
                return pltpu.make_async_copy(
                    table_sh.at[idx_v.at[pl.ds(off, sz)]],
                    rows.at[b, pl.ds(0, sz)],
                    sem_g.at[b],
                )

            def w_copy(kk):
                off, sz = blocks[kk]
                b = kk % NBUF
                return pltpu.make_async_copy(
                    rows.at[b, pl.ds(0, sz)],
                    out_hbm.at[pl.ds(base + off, sz)],
                    sem_w.at[b],
                )

            pltpu.sync_copy(idx_hbm.at[pl.ds(base, nrows)],
                            idx_v.at[pl.ds(0, nrows)])

            waited = []
            for kk in range(min(LOOK, K)):
                g_copy(kk).start()
            for kk in range(K):
                g_copy(kk).wait()
                w_copy(kk).start()
                nxt = kk + LOOK
                if nxt < K:
                    prev = nxt - NBUF
                    if prev >= 0:
                        w_copy(prev).wait()
                        waited.append(prev)
                    g_copy(nxt).start()
            for kk in range(K):
                if kk not in waited:
                    w_copy(kk).wait()

        @pl.when(wid < NW - 1)
        def _():
            ring(wid * B_MAIN, B_MAIN)

        @pl.when(wid == NW - 1)
        def _():
            ring(31 * B_MAIN, B_LAST)

    return k


_kernel = _make_kernel()


def kernel(atomic_num, embedding_table):
    idx = atomic_num.astype(jnp.int32)
    return _kernel(embedding_table, idx)


# speedup vs baseline: 5.8021x; 1.0220x over previous
import functools

import jax
import jax.numpy as jnp
from jax import lax
from jax.experimental import pallas as pl
from jax.experimental.pallas import tpu as pltpu
from jax.experimental.pallas import tpu_sc as plsc

N = 100000
D = 128
V = 95
NW = 32
B_MAIN = 3128
B_LAST = N - 31 * B_MAIN
BLK = 128
NBUF = 6
LOOK = 5
NF_MAIN = B_MAIN // BLK
NF_LAST = B_LAST // BLK
T_MAIN = B_MAIN - NF_MAIN * BLK
T_LAST = B_LAST - NF_LAST * BLK


def _make_kernel():
    mesh = plsc.VectorSubcoreMesh(core_axis_name="c", subcore_axis_name="s")

    @functools.partial(
        pl.kernel,
        mesh=mesh,
        out_type=jax.ShapeDtypeStruct((N, D), jnp.float32),
        scratch_types=[
            pltpu.VMEM_SHARED((V, D), jnp.float32),
            pltpu.VMEM((B_MAIN,), jnp.int32),
            pltpu.VMEM((NBUF, BLK, D), jnp.float32),
            pltpu.SemaphoreType.DMA((NBUF,)),
            pltpu.SemaphoreType.DMA((NBUF,)),
        ],
    )
    def k(table_hbm, idx_hbm, out_hbm, table_sh, idx_v, rows, sem_g, sem_w):
        cid = lax.axis_index("c")
        sid = lax.axis_index("s")
        wid = sid * 2 + cid
        last = wid == NW - 1
        base = wid * B_MAIN
        nfull = jnp.where(last, NF_LAST, NF_MAIN)

        @pl.when(sid == 0)
        def _():
            pltpu.sync_copy(table_hbm, table_sh)

        plsc.subcore_barrier()

        @pl.when(jnp.logical_not(last))
        def _():
            pltpu.sync_copy(idx_hbm.at[pl.ds(base, B_MAIN)], idx_v)

        @pl.when(last)
        def _():
            pltpu.sync_copy(idx_hbm.at[pl.ds(31 * B_MAIN, B_LAST)],
                            idx_v.at[pl.ds(0, B_LAST)])

        def g_copy(j, b):
            return pltpu.make_async_copy(
                table_sh.at[idx_v.at[pl.ds(j * BLK, BLK)]],
                rows.at[b],
                sem_g.at[b],
            )

        def w_copy(j, b):
            return pltpu.make_async_copy(
                rows.at[b],
                out_hbm.at[pl.ds(base + j * BLK, BLK)],
                sem_w.at[b],
            )

        for kk in range(LOOK):
            g_copy(kk, kk).start()

        def body(j, carry):
            b = j % NBUF
            g_copy(j, b).wait()
            w_copy(j, b).start()
            nxt = j + LOOK

            @pl.when(nxt < nfull)
            def _():
                @pl.when(j >= 1)
                def _():
                    w_copy(j - 1, (j - 1) % NBUF).wait()

                g_copy(nxt, nxt % NBUF).start()

            return carry

        lax.fori_loop(0, nfull, body, 0)

        w_copy(nfull - NBUF, (nfull - NBUF) % NBUF).wait()

        def tail(toff, tsz):
            b = (toff // BLK) % NBUF
            pltpu.make_async_copy(
                table_sh.at[idx_v.at[pl.ds(toff, tsz)]],
                rows.at[b, pl.ds(0, tsz)],
                sem_g.at[b],
            ).start()
            pltpu.make_async_copy(
                table_sh.at[idx_v.at[pl.ds(toff, tsz)]],
                rows.at[b, pl.ds(0, tsz)],
                sem_g.at[b],
            ).wait()
            pltpu.make_async_copy(
                rows.at[b, pl.ds(0, tsz)],
                out_hbm.at[pl.ds(base + toff, tsz)],
                sem_w.at[b],
            ).start()
            pltpu.make_async_copy(
                rows.at[b, pl.ds(0, tsz)],
                out_hbm.at[pl.ds(base + toff, tsz)],
                sem_w.at[b],
            ).wait()

        @pl.when(jnp.logical_not(last))
        def _():
            tail(NF_MAIN * BLK, T_MAIN)

        @pl.when(last)
        def _():
            tail(NF_LAST * BLK, T_LAST)

        def drain(j, carry):
            w_copy(j, j % NBUF).wait()
            return carry

        lax.fori_loop(nfull - LOOK, nfull, drain, 0)

    return k


_kernel = _make_kernel()


def kernel(atomic_num, embedding_table):
    idx = atomic_num.astype(jnp.int32)
    return _kernel(embedding_table, idx)


# speedup vs baseline: 5.8888x; 1.0149x over previous
import functools

import jax
import jax.numpy as jnp
from jax import lax
from jax.experimental import pallas as pl
from jax.experimental.pallas import tpu as pltpu
from jax.experimental.pallas import tpu_sc as plsc

N = 100000
D = 128
V = 95
NW = 32
B_MAIN = 3128
B_LAST = N - 31 * B_MAIN
BLK = 128
NBUF = 7
LOOK = 6
NF_MAIN = B_MAIN // BLK
NF_LAST = B_LAST // BLK
T_MAIN = B_MAIN - NF_MAIN * BLK
T_LAST = B_LAST - NF_LAST * BLK


def _make_kernel():
    mesh = plsc.VectorSubcoreMesh(core_axis_name="c", subcore_axis_name="s")

    @functools.partial(
        pl.kernel,
        mesh=mesh,
        out_type=jax.ShapeDtypeStruct((N, D), jnp.float32),
        scratch_types=[
            pltpu.VMEM_SHARED((V, D), jnp.float32),
            pltpu.VMEM((B_MAIN,), jnp.int32),
            pltpu.VMEM((NBUF, BLK, D), jnp.float32),
            pltpu.SemaphoreType.DMA((NBUF,)),
            pltpu.SemaphoreType.DMA((NBUF,)),
            pltpu.SemaphoreType.DMA,
        ],
    )
    def k(table_hbm, idx_hbm, out_hbm, table_sh, idx_v, rows, sem_g, sem_w,
          sem_i):
        cid = lax.axis_index("c")
        sid = lax.axis_index("s")
        wid = sid * 2 + cid
        last = wid == NW - 1
        base = wid * B_MAIN
        nfull = jnp.where(last, NF_LAST, NF_MAIN)

        @pl.when(jnp.logical_not(last))
        def _():
            pltpu.make_async_copy(
                idx_hbm.at[pl.ds(base, B_MAIN)], idx_v, sem_i).start()

        @pl.when(last)
        def _():
            pltpu.make_async_copy(
                idx_hbm.at[pl.ds(31 * B_MAIN, B_LAST)],
                idx_v.at[pl.ds(0, B_LAST)], sem_i).start()

        @pl.when(sid == 0)
        def _():
            pltpu.sync_copy(table_hbm, table_sh)

        plsc.subcore_barrier()

        @pl.when(jnp.logical_not(last))
        def _():
            pltpu.make_async_copy(
                idx_hbm.at[pl.ds(base, B_MAIN)], idx_v, sem_i).wait()

        @pl.when(last)
        def _():
            pltpu.make_async_copy(
                idx_hbm.at[pl.ds(31 * B_MAIN, B_LAST)],
                idx_v.at[pl.ds(0, B_LAST)], sem_i).wait()

        def g_copy(j, b):
            return pltpu.make_async_copy(
                table_sh.at[idx_v.at[pl.ds(j * BLK, BLK)]],
                rows.at[b],
                sem_g.at[b],
            )

        def w_copy(j, b):
            return pltpu.make_async_copy(
                rows.at[b],
                out_hbm.at[pl.ds(base + j * BLK, BLK)],
                sem_w.at[b],
            )

        for kk in range(LOOK):
            g_copy(kk, kk).start()

        def body(j, carry):
            b = j % NBUF
            g_copy(j, b).wait()
            w_copy(j, b).start()
            nxt = j + LOOK

            @pl.when(nxt < nfull)
            def _():
                @pl.when(j >= 1)
                def _():
                    w_copy(j - 1, (j - 1) % NBUF).wait()

                g_copy(nxt, nxt % NBUF).start()

            return carry

        lax.fori_loop(0, nfull, body, 0)

        w_copy(nfull - NBUF, (nfull - NBUF) % NBUF).wait()

        def tail(toff, tsz):
            b = (toff // BLK) % NBUF
            pltpu.make_async_copy(
                table_sh.at[idx_v.at[pl.ds(toff, tsz)]],
                rows.at[b, pl.ds(0, tsz)],
                sem_g.at[b],
            ).start()
            pltpu.make_async_copy(
                table_sh.at[idx_v.at[pl.ds(toff, tsz)]],
                rows.at[b, pl.ds(0, tsz)],
                sem_g.at[b],
            ).wait()
            pltpu.make_async_copy(
                rows.at[b, pl.ds(0, tsz)],
                out_hbm.at[pl.ds(base + toff, tsz)],
                sem_w.at[b],
            ).start()
            pltpu.make_async_copy(
                rows.at[b, pl.ds(0, tsz)],
                out_hbm.at[pl.ds(base + toff, tsz)],
                sem_w.at[b],
            ).wait()

        @pl.when(jnp.logical_not(last))
        def _():
            tail(NF_MAIN * BLK, T_MAIN)

        @pl.when(last)
        def _():
            tail(NF_LAST * BLK, T_LAST)

        def drain(j, carry):
            w_copy(j, j % NBUF).wait()
            return carry

        lax.fori_loop(nfull - LOOK, nfull, drain, 0)

    return k


_kernel = _make_kernel()


def kernel(atomic_num, embedding_table):
    idx = atomic_num.astype(jnp.int32)
    return _kernel(embedding_table, idx)
